# baseline (speedup 1.0000x reference)
import functools
import math

import jax
import jax.numpy as jnp
from jax import lax
from jax.experimental import pallas as pl
from jax.experimental.pallas import tpu as pltpu
from jax.experimental.pallas import tpu_sc as plsc

NUM_TOKENS = 16384 * 50
D_MODEL = 64
SCALE = math.sqrt(D_MODEL)

NC, NS, LANES = 2, 16, 16
NW = NC * NS
TOK_PER_W = NUM_TOKENS // NW
CHUNK = 512
SUB = 128
NSUB = CHUNK // SUB
NCHUNK = TOK_PER_W // CHUNK
IDXROWS_W = TOK_PER_W // SUB


def _emb_body(table_hbm, idx_hbm, out_hbm, idx_all, rows, gsem):
    wid = lax.axis_index("s") * NC + lax.axis_index("c")
    base = wid * TOK_PER_W
    idx_row0 = wid * IDXROWS_W

    pltpu.sync_copy(idx_hbm.at[pl.ds(idx_row0, IDXROWS_W)], idx_all)

    def chunk_body(c, carry):
        c4 = c * NSUB
        copies = []
        for j in range(NSUB):
            copies.append(
                pltpu.async_copy(
                    table_hbm.at[idx_all.at[c4 + j]],
                    rows.at[pl.ds(j * SUB, SUB)],
                    gsem,
                )
            )
        for cp in copies:
            cp.wait()

        gpr = SUB // LANES

        def gbody(g, _):
            idx16 = idx_all[c4 + g // gpr, pl.ds((g % gpr) * LANES, LANES)]
            sc16 = jnp.where(idx16 == 0, 0.0, SCALE).astype(jnp.float32)
            r0 = g * LANES
            for i in range(LANES):
                s = sc16[i]
                for cc in range(D_MODEL // LANES):
                    sl = pl.ds(cc * LANES, LANES)
                    rows[r0 + i, sl] = rows[r0 + i, sl] * s
            return 0

        lax.fori_loop(0, CHUNK // LANES, gbody, 0)

        pltpu.sync_copy(rows, out_hbm.at[pl.ds(base + c * CHUNK, CHUNK)])
        return carry

    lax.fori_loop(0, NCHUNK, chunk_body, 0)


_emb_kernel = functools.partial(
    pl.kernel,
    mesh=plsc.VectorSubcoreMesh(core_axis_name="c", subcore_axis_name="s"),
    out_type=jax.ShapeDtypeStruct((NUM_TOKENS, D_MODEL), jnp.float32),
    compiler_params=pltpu.CompilerParams(use_tc_tiling_on_sc=False),
    scratch_types=[
        pltpu.VMEM((IDXROWS_W, SUB), jnp.int32),
        pltpu.VMEM((CHUNK, D_MODEL), jnp.float32),
        pltpu.SemaphoreType.DMA,
    ],
)(_emb_body)


def kernel(inputs, table):
    b, t = inputs.shape
    idx2d = inputs.reshape(NUM_TOKENS // SUB, SUB).astype(jnp.int32)
    out = _emb_kernel(table, idx2d)
    return out.reshape(b, t, D_MODEL)


# speedup vs baseline: 1.0652x; 1.0652x over previous
import functools
import math

import jax
import jax.numpy as jnp
from jax import lax
from jax.experimental import pallas as pl
from jax.experimental.pallas import tpu as pltpu
from jax.experimental.pallas import tpu_sc as plsc

NUM_TOKENS = 16384 * 50
D_MODEL = 64
SCALE = math.sqrt(D_MODEL)

NC, NS, LANES = 2, 16, 16
NW = NC * NS
TOK_PER_W = NUM_TOKENS // NW
CHUNK = 256
SUB = 128
NSUB = CHUNK // SUB
NCHUNK = TOK_PER_W // CHUNK
IDXROWS_W = TOK_PER_W // SUB


def _emb_body(table_hbm, idx_hbm, out_hbm,
              idx_all, rows0, rows1, gsem0, gsem1, ssem0, ssem1):
    rows = (rows0, rows1)
    gsem = (gsem0, gsem1)
    ssem = (ssem0, ssem1)

    wid = lax.axis_index("s") * NC + lax.axis_index("c")
    base = wid * TOK_PER_W
    idx_row0 = wid * IDXROWS_W

    pltpu.sync_copy(idx_hbm.at[pl.ds(idx_row0, IDXROWS_W)], idx_all)

    def fire_gathers(chunk, buf):
        for j in range(NSUB):
            pltpu.async_copy(
                table_hbm.at[idx_all.at[chunk * NSUB + j]],
                rows[buf].at[pl.ds(j * SUB, SUB)],
                gsem[buf],
            )

    def wait_gathers(buf):
        for j in range(NSUB):
            pltpu.make_async_copy(
                table_hbm.at[idx_all.at[j]],
                rows[buf].at[pl.ds(j * SUB, SUB)],
                gsem[buf],
            ).wait()

    def fire_store(chunk, buf):
        pltpu.async_copy(
            rows[buf], out_hbm.at[pl.ds(base + chunk * CHUNK, CHUNK)],
            ssem[buf],
        )

    def wait_store(buf):
        pltpu.make_async_copy(
            rows[buf], out_hbm.at[pl.ds(base, CHUNK)], ssem[buf],
        ).wait()

    def scale_chunk(chunk, buf):
        gpr = SUB // LANES

        def gbody(g, _):
            idx16 = idx_all[chunk * NSUB + g // gpr,
                            pl.ds((g % gpr) * LANES, LANES)]
            sc16 = jnp.where(idx16 == 0, 0.0, SCALE).astype(jnp.float32)
            r0 = g * LANES
            for i in range(LANES):
                s = sc16[i]
                for cc in range(D_MODEL // LANES):
                    sl = pl.ds(cc * LANES, LANES)
                    rows[buf][r0 + i, sl] = rows[buf][r0 + i, sl] * s
            return 0

        lax.fori_loop(0, CHUNK // LANES, gbody, 0)

    fire_gathers(0, 0)

    @pl.loop(0, NCHUNK, step=2)
    def _pipeline(c):
        for b in range(2):
            cc = c + b
            nb = 1 - b
            @pl.when(cc >= 1)
            def _():
                wait_store(nb)

            nxt = jnp.minimum(cc + 1, NCHUNK - 1)
            fire_gathers(nxt, nb)

            wait_gathers(b)
            scale_chunk(cc, b)
            fire_store(cc, b)

    wait_store((NCHUNK - 1) % 2)
    wait_gathers((NCHUNK) % 2)


_emb_kernel = functools.partial(
    pl.kernel,
    mesh=plsc.VectorSubcoreMesh(core_axis_name="c", subcore_axis_name="s"),
    out_type=jax.ShapeDtypeStruct((NUM_TOKENS, D_MODEL), jnp.float32),
    compiler_params=pltpu.CompilerParams(use_tc_tiling_on_sc=False),
    scratch_types=[
        pltpu.VMEM((IDXROWS_W, SUB), jnp.int32),
        pltpu.VMEM((CHUNK, D_MODEL), jnp.float32),
        pltpu.VMEM((CHUNK, D_MODEL), jnp.float32),
        pltpu.SemaphoreType.DMA,
        pltpu.SemaphoreType.DMA,
        pltpu.SemaphoreType.DMA,
        pltpu.SemaphoreType.DMA,
    ],
)(_emb_body)


def kernel(inputs, table):
    b, t = inputs.shape
    idx2d = inputs.reshape(NUM_TOKENS // SUB, SUB).astype(jnp.int32)
    out = _emb_kernel(table, idx2d)
    return out.reshape(b, t, D_MODEL)


# speedup vs baseline: 1.0691x; 1.0036x over previous
import functools
import math

import jax
import jax.numpy as jnp
from jax import lax
from jax.experimental import pallas as pl
from jax.experimental.pallas import tpu as pltpu
from jax.experimental.pallas import tpu_sc as plsc

NUM_TOKENS = 16384 * 50
D_MODEL = 64
SCALE = math.sqrt(D_MODEL)

NC, NS, LANES = 2, 16, 16
NW = NC * NS
TOK_PER_W = NUM_TOKENS // NW
CHUNK = 256
SUB = 128
NSUB = CHUNK // SUB
NCHUNK = TOK_PER_W // CHUNK
IDXROWS_W = TOK_PER_W // SUB


def _emb_body(table_hbm, idx_hbm, out_hbm,
              idx_all, rowsi0, rowsi1, rowso0, rowso1,
              gsem0, gsem1, ssem0, ssem1):
    rowsi = (rowsi0, rowsi1)
    rowso = (rowso0, rowso1)
    gsem = (gsem0, gsem1)
    ssem = (ssem0, ssem1)

    wid = lax.axis_index("s") * NC + lax.axis_index("c")
    base = wid * TOK_PER_W
    idx_row0 = wid * IDXROWS_W

    pltpu.sync_copy(idx_hbm.at[pl.ds(idx_row0, IDXROWS_W)], idx_all)

    def fire_gathers(chunk, buf):
        for j in range(NSUB):
            pltpu.async_copy(
                table_hbm.at[idx_all.at[chunk * NSUB + j]],
                rowsi[buf].at[pl.ds(j * SUB, SUB)],
                gsem[buf],
            )

    def wait_gathers(buf):
        for j in range(NSUB):
            pltpu.make_async_copy(
                table_hbm.at[idx_all.at[j]],
                rowsi[buf].at[pl.ds(j * SUB, SUB)],
                gsem[buf],
            ).wait()

    def fire_store(chunk, buf):
        pltpu.async_copy(
            rowso[buf], out_hbm.at[pl.ds(base + chunk * CHUNK, CHUNK)],
            ssem[buf],
        )

    def wait_store(buf):
        pltpu.make_async_copy(
            rowso[buf], out_hbm.at[pl.ds(base, CHUNK)], ssem[buf],
        ).wait()

    def scale_chunk(chunk, buf):
        gpr = SUB // LANES

        def gbody(g, _):
            idx16 = idx_all[chunk * NSUB + g // gpr,
                            pl.ds((g % gpr) * LANES, LANES)]
            sc16 = jnp.where(idx16 == 0, 0.0, SCALE).astype(jnp.float32)
            r0 = g * LANES
            for i in range(LANES):
                s = sc16[i]
                for cc in range(D_MODEL // LANES):
                    sl = pl.ds(cc * LANES, LANES)
                    rowso[buf][r0 + i, sl] = rowsi[buf][r0 + i, sl] * s
            return 0

        lax.fori_loop(0, CHUNK // LANES, gbody, 0)

    fire_gathers(0, 0)

    @pl.loop(0, NCHUNK, step=2)
    def _pipeline(c):
        for b in range(2):
            cc = c + b
            nb = 1 - b
            nxt = jnp.minimum(cc + 1, NCHUNK - 1)
            fire_gathers(nxt, nb)

            wait_gathers(b)
            @pl.when(cc >= 2)
            def _():
                wait_store(b)

            scale_chunk(cc, b)
            fire_store(cc, b)

    wait_store(0)
    wait_store(1)
    wait_gathers(NCHUNK % 2)


_emb_kernel = functools.partial(
    pl.kernel,
    mesh=plsc.VectorSubcoreMesh(core_axis_name="c", subcore_axis_name="s"),
    out_type=jax.ShapeDtypeStruct((NUM_TOKENS, D_MODEL), jnp.float32),
    compiler_params=pltpu.CompilerParams(use_tc_tiling_on_sc=False),
    scratch_types=[
        pltpu.VMEM((IDXROWS_W, SUB), jnp.int32),
        pltpu.VMEM((CHUNK, D_MODEL), jnp.float32),
        pltpu.VMEM((CHUNK, D_MODEL), jnp.float32),
        pltpu.VMEM((CHUNK, D_MODEL), jnp.float32),
        pltpu.VMEM((CHUNK, D_MODEL), jnp.float32),
        pltpu.SemaphoreType.DMA,
        pltpu.SemaphoreType.DMA,
        pltpu.SemaphoreType.DMA,
        pltpu.SemaphoreType.DMA,
    ],
)(_emb_body)


def kernel(inputs, table):
    b, t = inputs.shape
    idx2d = inputs.reshape(NUM_TOKENS // SUB, SUB).astype(jnp.int32)
    out = _emb_kernel(table, idx2d)
    return out.reshape(b, t, D_MODEL)
